# baseline (speedup 1.0000x reference)
import functools

import jax
import jax.numpy as jnp
from jax import lax
from jax.experimental import pallas as pl
from jax.experimental.pallas import tpu as pltpu
from jax.experimental.pallas import tpu_sc as plsc

VOCAB = 128
SEQ = 72
D = 1024
BATCH = 4096

NC = 2
NS = 16
NW = NC * NS
LANES = 16

NTOK = BATCH * SEQ
TOK_PER_W = NTOK // NW
CHUNK = 16
N_CHUNKS = TOK_PER_W // CHUNK
NROW = 3
NPOS = 2
NIDX = 3
TROWS = VOCAB + SEQ + CHUNK

STEADY_LO = 2
STEADY_N = (N_CHUNKS - 4 - STEADY_LO) // 6
assert STEADY_LO + 6 * STEADY_N == N_CHUNKS - 4
assert CHUNK == LANES and TOK_PER_W % SEQ == 0

_MESH = plsc.VectorSubcoreMesh(core_axis_name="c", subcore_axis_name="s")


@functools.partial(
    pl.kernel,
    out_type=jax.ShapeDtypeStruct((NTOK, D), jnp.float32),
    mesh=_MESH,
    scratch_types=[
        pltpu.VMEM_SHARED((TROWS, D), jnp.float32),
        [pltpu.VMEM((CHUNK, D), jnp.float32) for _ in range(NROW)],
        [pltpu.VMEM((CHUNK, D), jnp.float32) for _ in range(NPOS)],
        pltpu.VMEM((NIDX, CHUNK), jnp.int32),
        [pltpu.SemaphoreType.DMA for _ in range(NROW)],
        [pltpu.SemaphoreType.DMA for _ in range(NPOS)],
        [pltpu.SemaphoreType.DMA for _ in range(NIDX)],
        [pltpu.SemaphoreType.DMA for _ in range(NROW)],
    ],
)
def _sc_kernel(idx_hbm, tok_hbm, pos_hbm, out_hbm,
               table, rows, posb, idxr,
               gsems, psems, isems, ssems):
    cid = lax.axis_index("c")
    sid = lax.axis_index("s")
    wid = sid * NC + cid
    base = wid * TOK_PER_W
    row0 = wid * N_CHUNKS

    pltpu.sync_copy(tok_hbm.at[pl.ds(sid * 8, 8)],
                    table.at[pl.ds(sid * 8, 8)])

    @pl.when(sid < 9)
    def _():
        pltpu.sync_copy(pos_hbm.at[pl.ds(sid * 8, 8)],
                        table.at[pl.ds(VOCAB + sid * 8, 8)])

    @pl.when((sid >= 9) & (sid < 11))
    def _():
        pltpu.sync_copy(pos_hbm.at[pl.ds((sid - 9) * 8, 8)],
                        table.at[pl.ds(VOCAB + SEQ + (sid - 9) * 8, 8)])

    plsc.subcore_barrier()

    def fire_idx(j, sl):
        pltpu.async_copy(idx_hbm.at[pl.ds((row0 + j) * CHUNK, CHUNK)],
                         idxr.at[sl], isems[sl])

    def wait_idx(j, sl):
        pltpu.make_async_copy(idx_hbm.at[pl.ds((row0 + j) * CHUNK, CHUNK)],
                              idxr.at[sl], isems[sl]).wait()

    def fire_g(j, sl):
        xv = idxr[sl]
        for t in range(CHUNK):
            pltpu.async_copy(table.at[xv[t]], rows[sl].at[t], gsems[sl])

    def wait_g(j, sl):
        pltpu.make_async_copy(out_hbm.at[pl.ds(base, CHUNK)], rows[sl],
                              gsems[sl]).wait()

    def fire_p(j, sl):
        s0 = VOCAB + lax.rem(j * CHUNK, SEQ)
        pltpu.async_copy(table.at[pl.ds(s0, CHUNK)], posb[sl], psems[sl])

    def wait_p(j, sl):
        pltpu.make_async_copy(out_hbm.at[pl.ds(base, CHUNK)], posb[sl],
                              psems[sl]).wait()

    def fire_s(i, sl):
        pltpu.async_copy(rows[sl], out_hbm.at[pl.ds(base + i * CHUNK, CHUNK)],
                         ssems[sl])

    def wait_s(i, sl):
        pltpu.make_async_copy(rows[sl],
                              out_hbm.at[pl.ds(base + i * CHUNK, CHUNK)],
                              ssems[sl]).wait()

    def addpass(b3, b2):
        def per_tok(t, carry):
            def per_vec(c, carry2):
                sl = (t, pl.ds(c * LANES, LANES))
                plsc.addupdate(rows[b3].at[sl], posb[b2][sl])
                return carry2

            return lax.fori_loop(0, D // LANES, per_vec, carry, unroll=8)

        lax.fori_loop(0, CHUNK, per_tok, 0)

    def pipe_iter(i, jm, do_ws=True, do_gp=True, do_fi=True):
        s_row, s_pos = jm % NROW, jm % NPOS
        s_row_n, s_pos_n = (jm + 1) % NROW, (jm + 1) % NPOS
        s_idx_n = (jm + 1) % NIDX
        if do_ws:
            wait_s(i - 2, s_row_n)
        if do_gp:
            wait_idx(i + 1, s_idx_n)
            fire_g(i + 1, s_row_n)
            fire_p(i + 1, s_pos_n)
        wait_g(i, s_row)
        wait_p(i, s_pos)
        if do_fi:
            fire_idx(i + 3, jm % NIDX)
        addpass(s_row, s_pos)
        fire_s(i, s_row)

    fire_idx(0, 0)
    fire_idx(1, 1)
    fire_idx(2, 2)
    wait_idx(0, 0)
    fire_g(0, 0)
    fire_p(0, 0)
    for i in range(STEADY_LO):
        pipe_iter(i, i, do_ws=False)

    def step(k, carry):
        for jj in range(6):
            pipe_iter(STEADY_LO + k * 6 + jj, STEADY_LO + jj)
        return carry

    lax.fori_loop(0, STEADY_N, step, 0)

    for i in range(N_CHUNKS - 4, N_CHUNKS):
        pipe_iter(i, i,
                  do_gp=i + 1 <= N_CHUNKS - 1,
                  do_fi=i + 3 <= N_CHUNKS - 1)
    wait_s(N_CHUNKS - 2, (N_CHUNKS - 2) % NROW)
    wait_s(N_CHUNKS - 1, (N_CHUNKS - 1) % NROW)


def kernel(x, tok_embed, pos_embed):
    pos2d = pos_embed.reshape(SEQ, D).astype(jnp.float32)
    x1d = x.reshape(NTOK).astype(jnp.int32)
    out = _sc_kernel(x1d, tok_embed.astype(jnp.float32), pos2d)
    return out.reshape(BATCH, SEQ, D)


# speedup vs baseline: 1.1852x; 1.1852x over previous
import functools

import jax
import jax.numpy as jnp
from jax import lax
from jax.experimental import pallas as pl
from jax.experimental.pallas import tpu as pltpu
from jax.experimental.pallas import tpu_sc as plsc

VOCAB = 128
SEQ = 72
D = 1024
DH = D // 2
BATCH = 4096

NC = 2
NS = 16
NW = NC * NS
LANES = 16

NTOK = BATCH * SEQ
TOK_PER_W = NTOK // NW
CHUNK = 16
N_CHUNKS = TOK_PER_W // CHUNK
S_BLK = 8

STEADY_LO = 2
STEADY_N = (N_CHUNKS - 2 - STEADY_LO) // 2
assert STEADY_LO + 2 * STEADY_N == N_CHUNKS - 2
assert CHUNK == LANES


def _shuffle_pairs(a):
    n = a.shape[-1]
    return (
        a.reshape(a.shape[:-1] + (n // 32, 2, 16))
        .swapaxes(-2, -1)
        .reshape(a.shape[:-1] + (n,))
    )


def _fused_body(tok_ref, pos_ref, out_ref):
    s = tok_ref[...][None, :, :] + pos_ref[...][:, None, :]
    out_ref[...] = s.astype(jnp.bfloat16)


def _build_fused(tok_embed, pos2d):
    return pl.pallas_call(
        _fused_body,
        grid=(SEQ // S_BLK,),
        in_specs=[
            pl.BlockSpec((VOCAB, D), lambda s: (0, 0)),
            pl.BlockSpec((S_BLK, D), lambda s: (s, 0)),
        ],
        out_specs=pl.BlockSpec((S_BLK, VOCAB, D), lambda s: (s, 0, 0)),
        out_shape=jax.ShapeDtypeStruct((SEQ, VOCAB, D), jnp.bfloat16),
    )(tok_embed, pos2d)


_MESH = plsc.VectorSubcoreMesh(core_axis_name="c", subcore_axis_name="s")


@functools.partial(
    pl.kernel,
    out_type=jax.ShapeDtypeStruct((NTOK, D), jnp.int32),
    mesh=_MESH,
    scratch_types=[
        [pltpu.VMEM((CHUNK, DH), jnp.int32) for _ in range(2)],
        [pltpu.VMEM((CHUNK, D), jnp.int32) for _ in range(2)],
        pltpu.VMEM((2, CHUNK), jnp.int32),
        [pltpu.SemaphoreType.DMA for _ in range(2)],
        [pltpu.SemaphoreType.DMA for _ in range(2)],
        [pltpu.SemaphoreType.DMA for _ in range(2)],
    ],
)
def _sc_kernel(idx_hbm, fused_hbm, out_hbm,
               rowsbf, outb, idxr, gsems, isems, ssems):
    cid = lax.axis_index("c")
    sid = lax.axis_index("s")
    wid = sid * NC + cid
    base = wid * TOK_PER_W

    def fire_idx(j, sl):
        pltpu.async_copy(idx_hbm.at[pl.ds((base // CHUNK + j) * CHUNK, CHUNK)],
                         idxr.at[sl], isems[sl])

    def wait_idx(j, sl):
        pltpu.make_async_copy(
            idx_hbm.at[pl.ds((base // CHUNK + j) * CHUNK, CHUNK)],
            idxr.at[sl], isems[sl]).wait()

    def fuse(j, sl):
        p = base + j * CHUNK + lax.iota(jnp.int32, LANES)
        idxr[sl, :] = lax.rem(p, SEQ) * VOCAB + idxr[sl, :]

    def fire_g(j, sl):
        pltpu.async_copy(fused_hbm.at[idxr.at[sl]], rowsbf[sl], gsems[sl])

    def wait_g(j, sl):
        pltpu.make_async_copy(fused_hbm.at[idxr.at[sl]], rowsbf[sl],
                              gsems[sl]).wait()

    def fire_s(i, sl):
        pltpu.async_copy(outb[sl], out_hbm.at[pl.ds(base + i * CHUNK, CHUNK)],
                         ssems[sl])

    def wait_s(i, sl):
        pltpu.make_async_copy(outb[sl],
                              out_hbm.at[pl.ds(base + i * CHUNK, CHUNK)],
                              ssems[sl]).wait()

    def conv(b):
        def per_tok(t, carry):
            def per_vec(c, carry2):
                w = rowsbf[b][t, pl.ds(c * LANES, LANES)]
                sixteen = jnp.full((LANES,), 16, jnp.int32)
                mask = jnp.full((LANES,), -65536, jnp.int32)
                lo = lax.shift_left(w, sixteen)
                hi = lax.bitwise_and(w, mask)
                outb[b][t, pl.ds(2 * c * LANES, LANES)] = lo
                outb[b][t, pl.ds((2 * c + 1) * LANES, LANES)] = hi
                return carry2

            return lax.fori_loop(0, DH // LANES, per_vec, carry, unroll=8)

        lax.fori_loop(0, CHUNK, per_tok, 0)

    def pipe_iter(i, jm, do_ws=True, do_g=True, do_fi=True):
        b, bn = jm % 2, (jm + 1) % 2
        wait_g(i, b)
        if do_fi:
            fire_idx(i + 2, b)
        if do_g:
            wait_idx(i + 1, bn)
            fuse(i + 1, bn)
            fire_g(i + 1, bn)
        if do_ws:
            wait_s(i - 2, b)
        conv(b)
        fire_s(i, b)

    fire_idx(0, 0)
    fire_idx(1, 1)
    wait_idx(0, 0)
    fuse(0, 0)
    fire_g(0, 0)
    for i in range(STEADY_LO):
        pipe_iter(i, i, do_ws=False)

    def step(k, carry):
        for jj in range(2):
            pipe_iter(STEADY_LO + k * 2 + jj, STEADY_LO + jj,
                      do_fi=True)
        return carry

    lax.fori_loop(0, STEADY_N, step, 0)

    for i in range(N_CHUNKS - 2, N_CHUNKS):
        pipe_iter(i, i,
                  do_g=i + 1 <= N_CHUNKS - 1,
                  do_fi=i + 2 <= N_CHUNKS - 1)
    wait_s(N_CHUNKS - 2, (N_CHUNKS - 2) % 2)
    wait_s(N_CHUNKS - 1, (N_CHUNKS - 1) % 2)


def kernel(x, tok_embed, pos_embed):
    tok_s = _shuffle_pairs(tok_embed.astype(jnp.float32))
    pos_s = _shuffle_pairs(pos_embed.reshape(SEQ, D).astype(jnp.float32))
    fused_bf = _build_fused(tok_s, pos_s)
    fused_i32 = lax.bitcast_convert_type(
        fused_bf.reshape(SEQ * VOCAB, DH, 2), jnp.int32)
    x1d = x.reshape(NTOK).astype(jnp.int32)
    out = _sc_kernel(x1d, fused_i32)
    return lax.bitcast_convert_type(out, jnp.float32).reshape(BATCH, SEQ, D)


# speedup vs baseline: 1.1853x; 1.0001x over previous
import functools

import jax
import jax.numpy as jnp
from jax import lax
from jax.experimental import pallas as pl
from jax.experimental.pallas import tpu as pltpu
from jax.experimental.pallas import tpu_sc as plsc

VOCAB = 128
SEQ = 72
D = 1024
DH = D // 2
BATCH = 4096

NC = 2
NS = 16
NW = NC * NS
LANES = 16

NTOK = BATCH * SEQ
TOK_PER_W = NTOK // NW
CHUNK = 16
N_CHUNKS = TOK_PER_W // CHUNK
S_BLK = 8

STEADY_LO = 2
STEADY_N = (N_CHUNKS - 2 - STEADY_LO) // 2
assert STEADY_LO + 2 * STEADY_N == N_CHUNKS - 2
assert CHUNK == LANES


def _shuffle_pairs(a):
    n = a.shape[-1]
    return (
        a.reshape(a.shape[:-1] + (n // 32, 2, 16))
        .swapaxes(-2, -1)
        .reshape(a.shape[:-1] + (n,))
    )


def _fused_body(tok_ref, pos_ref, out_ref):
    s = tok_ref[...][None, :, :] + pos_ref[...][:, None, :]
    out_ref[...] = s.astype(jnp.bfloat16)


def _build_fused(tok_embed, pos2d):
    return pl.pallas_call(
        _fused_body,
        grid=(SEQ // S_BLK,),
        in_specs=[
            pl.BlockSpec((VOCAB, D), lambda s: (0, 0)),
            pl.BlockSpec((S_BLK, D), lambda s: (s, 0)),
        ],
        out_specs=pl.BlockSpec((S_BLK, VOCAB, D), lambda s: (s, 0, 0)),
        out_shape=jax.ShapeDtypeStruct((SEQ, VOCAB, D), jnp.bfloat16),
    )(tok_embed, pos2d)


_MESH = plsc.VectorSubcoreMesh(core_axis_name="c", subcore_axis_name="s")


@functools.partial(
    pl.kernel,
    out_type=jax.ShapeDtypeStruct((NTOK, D), jnp.int32),
    mesh=_MESH,
    scratch_types=[
        [pltpu.VMEM((CHUNK, DH), jnp.int32) for _ in range(2)],
        [pltpu.VMEM((CHUNK, D), jnp.int32) for _ in range(2)],
        pltpu.VMEM((2, CHUNK), jnp.int32),
        [pltpu.SemaphoreType.DMA for _ in range(2)],
        [pltpu.SemaphoreType.DMA for _ in range(2)],
        [pltpu.SemaphoreType.DMA for _ in range(2)],
    ],
)
def _sc_kernel(idx_hbm, fused_hbm, out_hbm,
               rowsbf, outb, idxr, gsems, isems, ssems):
    cid = lax.axis_index("c")
    sid = lax.axis_index("s")
    wid = sid * NC + cid
    base = wid * TOK_PER_W

    def fire_idx(j, sl):
        pltpu.async_copy(idx_hbm.at[pl.ds((base // CHUNK + j) * CHUNK, CHUNK)],
                         idxr.at[sl], isems[sl])

    def wait_idx(j, sl):
        pltpu.make_async_copy(
            idx_hbm.at[pl.ds((base // CHUNK + j) * CHUNK, CHUNK)],
            idxr.at[sl], isems[sl]).wait()

    def fuse(j, sl):
        p = base + j * CHUNK + lax.iota(jnp.int32, LANES)
        idxr[sl, :] = lax.rem(p, SEQ) * VOCAB + idxr[sl, :]

    def fire_g(j, sl):
        pltpu.async_copy(fused_hbm.at[idxr.at[sl]], rowsbf[sl], gsems[sl])

    def wait_g(j, sl):
        pltpu.make_async_copy(fused_hbm.at[idxr.at[sl]], rowsbf[sl],
                              gsems[sl]).wait()

    def fire_s(i, sl):
        pltpu.async_copy(outb[sl], out_hbm.at[pl.ds(base + i * CHUNK, CHUNK)],
                         ssems[sl])

    def wait_s(i, sl):
        pltpu.make_async_copy(outb[sl],
                              out_hbm.at[pl.ds(base + i * CHUNK, CHUNK)],
                              ssems[sl]).wait()

    sixteen = jnp.full((LANES,), 16, jnp.int32)
    mask = jnp.full((LANES,), -65536, jnp.int32)

    def conv(b):
        def per_tok(t, carry):
            for c in range(DH // LANES):
                w = rowsbf[b][t, pl.ds(c * LANES, LANES)]
                outb[b][t, pl.ds(2 * c * LANES, LANES)] = (
                    lax.shift_left(w, sixteen))
                outb[b][t, pl.ds((2 * c + 1) * LANES, LANES)] = (
                    lax.bitwise_and(w, mask))
            return carry

        lax.fori_loop(0, CHUNK, per_tok, 0)

    def pipe_iter(i, jm, do_ws=True, do_g=True, do_fi=True):
        b, bn = jm % 2, (jm + 1) % 2
        wait_g(i, b)
        if do_fi:
            fire_idx(i + 2, b)
        if do_g:
            wait_idx(i + 1, bn)
            fuse(i + 1, bn)
            fire_g(i + 1, bn)
        if do_ws:
            wait_s(i - 2, b)
        conv(b)
        fire_s(i, b)

    fire_idx(0, 0)
    fire_idx(1, 1)
    wait_idx(0, 0)
    fuse(0, 0)
    fire_g(0, 0)
    for i in range(STEADY_LO):
        pipe_iter(i, i, do_ws=False)

    def step(k, carry):
        for jj in range(2):
            pipe_iter(STEADY_LO + k * 2 + jj, STEADY_LO + jj,
                      do_fi=True)
        return carry

    lax.fori_loop(0, STEADY_N, step, 0)

    for i in range(N_CHUNKS - 2, N_CHUNKS):
        pipe_iter(i, i,
                  do_g=i + 1 <= N_CHUNKS - 1,
                  do_fi=i + 2 <= N_CHUNKS - 1)
    wait_s(N_CHUNKS - 2, (N_CHUNKS - 2) % 2)
    wait_s(N_CHUNKS - 1, (N_CHUNKS - 1) % 2)


def kernel(x, tok_embed, pos_embed):
    tok_s = _shuffle_pairs(tok_embed.astype(jnp.float32))
    pos_s = _shuffle_pairs(pos_embed.reshape(SEQ, D).astype(jnp.float32))
    fused_bf = _build_fused(tok_s, pos_s)
    fused_i32 = lax.bitcast_convert_type(
        fused_bf.reshape(SEQ * VOCAB, DH, 2), jnp.int32)
    x1d = x.reshape(NTOK).astype(jnp.int32)
    out = _sc_kernel(x1d, fused_i32)
    return lax.bitcast_convert_type(out, jnp.float32).reshape(BATCH, SEQ, D)
